# Initial kernel scaffold; baseline (speedup 1.0000x reference)
import jax
import jax.numpy as jnp
from jax.experimental import pallas as pl


def kernel(inputs, emb_table, pos_table, W1, b1, W2, b2):
    raise NotImplementedError("write your pallas kernel here")


# speedup vs baseline: 7.5401x; 7.5401x over previous
import functools

import jax
import jax.numpy as jnp
from jax import lax
from jax.experimental import pallas as pl
from jax.experimental.pallas import tpu as pltpu
from jax.experimental.pallas import tpu_sc as plsc

_VOCAB = 1000000
_D_EMBED = 32
_D_POS = 16

_NUM_WORKERS = 32
_ROWS_PER_DMA = 128
_CHUNK = 1024


def _sc_gather(idx_flat, emb_table, pos_table):
    n = idx_flat.shape[0]
    per_w = n // _NUM_WORKERS
    n_rounds = per_w // _CHUNK
    n_dma = _CHUNK // _ROWS_PER_DMA

    mesh = plsc.VectorSubcoreMesh(core_axis_name="c", subcore_axis_name="s")

    @functools.partial(
        pl.kernel,
        mesh=mesh,
        compiler_params=pltpu.CompilerParams(use_tc_tiling_on_sc=False),
        out_type=(
            jax.ShapeDtypeStruct((n, _D_EMBED), jnp.float32),
            jax.ShapeDtypeStruct((n, _D_POS), jnp.float32),
        ),
        scratch_types=[
            pltpu.VMEM((per_w,), jnp.int32),
            pltpu.VMEM((per_w,), jnp.int32),
            pltpu.VMEM((per_w,), jnp.int32),
            pltpu.VMEM((_CHUNK, _D_EMBED), jnp.float32),
            pltpu.VMEM((_CHUNK, _D_POS), jnp.float32),
            pltpu.SemaphoreType.DMA,
            pltpu.SemaphoreType.DMA,
        ],
    )
    def gather_kernel(idx_hbm, emb_hbm, pos_hbm, we_out, pe_out,
                      idx_v, word_v, posi_v, we_v, pe_v, sem_w, sem_p):
        wid = lax.axis_index("s") * 2 + lax.axis_index("c")
        base = wid * per_w
        pltpu.sync_copy(idx_hbm.at[pl.ds(base, per_w)], idx_v)

        vocab_c = jnp.full((16,), _VOCAB, jnp.int32)

        def split_body(i, carry):
            r = idx_v[pl.ds(i * 16, 16)]
            word_v[pl.ds(i * 16, 16)] = lax.rem(r, vocab_c)
            posi_v[pl.ds(i * 16, 16)] = lax.div(r, vocab_c)
            return carry

        lax.fori_loop(0, per_w // 16, split_body, 0)

        def round_body(r, carry):
            descs = []
            for b in range(n_dma):
                off = r * _CHUNK + b * _ROWS_PER_DMA
                descs.append(pltpu.async_copy(
                    emb_hbm.at[word_v.at[pl.ds(off, _ROWS_PER_DMA)]],
                    we_v.at[pl.ds(b * _ROWS_PER_DMA, _ROWS_PER_DMA)],
                    sem_w))
                descs.append(pltpu.async_copy(
                    pos_hbm.at[posi_v.at[pl.ds(off, _ROWS_PER_DMA)]],
                    pe_v.at[pl.ds(b * _ROWS_PER_DMA, _ROWS_PER_DMA)],
                    sem_p))
            for d in descs:
                d.wait()
            pltpu.sync_copy(we_v, we_out.at[pl.ds(base + r * _CHUNK, _CHUNK)])
            pltpu.sync_copy(pe_v, pe_out.at[pl.ds(base + r * _CHUNK, _CHUNK)])
            return carry

        lax.fori_loop(0, n_rounds, round_body, 0)

    return gather_kernel(idx_flat, emb_table, pos_table)


def _mlp(we, pe, W1, b1, W2, b2, block):
    n = we.shape[0]

    def body(we_ref, pe_ref, w1_ref, b1_ref, w2_ref, b2_ref, out_ref):
        w1 = w1_ref[...]
        h = jnp.dot(we_ref[...], w1[:_D_EMBED, :],
                    preferred_element_type=jnp.float32)
        h += jnp.dot(pe_ref[...], w1[_D_EMBED:, :],
                     preferred_element_type=jnp.float32)
        h = jnp.maximum(h + b1_ref[...], 0.0)
        out_ref[...] = jnp.dot(h, w2_ref[...],
                               preferred_element_type=jnp.float32) + b2_ref[...]

    return pl.pallas_call(
        body,
        grid=(n // block,),
        in_specs=[
            pl.BlockSpec((block, _D_EMBED), lambda i: (i, 0)),
            pl.BlockSpec((block, _D_POS), lambda i: (i, 0)),
            pl.BlockSpec((_D_EMBED + _D_POS, _D_EMBED), lambda i: (0, 0)),
            pl.BlockSpec((1, _D_EMBED), lambda i: (0, 0)),
            pl.BlockSpec((_D_EMBED, 64), lambda i: (0, 0)),
            pl.BlockSpec((1, 64), lambda i: (0, 0)),
        ],
        out_specs=pl.BlockSpec((block, 64), lambda i: (i, 0)),
        out_shape=jax.ShapeDtypeStruct((n, 64), jnp.float32),
    )(we, pe, W1, b1.reshape(1, -1), W2, b2.reshape(1, -1))


def kernel(inputs, emb_table, pos_table, W1, b1, W2, b2):
    b, l = inputs.shape
    idx = inputs.reshape(-1).astype(jnp.int32)
    we, pe = _sc_gather(idx, emb_table, pos_table)
    out = _mlp(we, pe, W1, b1, W2, b2, block=2048)
    return out.reshape(b, l, W2.shape[1])


# speedup vs baseline: 9.5459x; 1.2660x over previous
import functools

import jax
import jax.numpy as jnp
from jax import lax
from jax.experimental import pallas as pl
from jax.experimental.pallas import tpu as pltpu
from jax.experimental.pallas import tpu_sc as plsc

_VOCAB = 1000000
_D_EMBED = 32
_D_POS = 16

_NUM_WORKERS = 32
_ROWS_PER_DMA = 128
_CHUNK = 1024


def _sc_gather(idx_flat, emb_table, pos_pad):
    n = idx_flat.shape[0]
    per_w = n // _NUM_WORKERS
    n_rounds = per_w // _CHUNK
    n_dma = _CHUNK // _ROWS_PER_DMA

    mesh = plsc.VectorSubcoreMesh(core_axis_name="c", subcore_axis_name="s")

    @functools.partial(
        pl.kernel,
        mesh=mesh,
        compiler_params=pltpu.CompilerParams(use_tc_tiling_on_sc=False),
        out_type=(
            jax.ShapeDtypeStruct((n, _D_EMBED), jnp.float32),
            jax.ShapeDtypeStruct((n, _D_EMBED), jnp.float32),
        ),
        scratch_types=[
            pltpu.VMEM((per_w,), jnp.int32),
            pltpu.VMEM((per_w,), jnp.int32),
            pltpu.VMEM((per_w,), jnp.int32),
            pltpu.VMEM((_CHUNK, _D_EMBED), jnp.float32),
            pltpu.VMEM((_CHUNK, _D_EMBED), jnp.float32),
            pltpu.SemaphoreType.DMA,
            pltpu.SemaphoreType.DMA,
        ],
    )
    def gather_kernel(idx_hbm, emb_hbm, pos_hbm, we_out, pe_out,
                      idx_v, word_v, posi_v, we_v, pe_v, sem_w, sem_p):
        wid = lax.axis_index("s") * 2 + lax.axis_index("c")
        base = wid * per_w
        pltpu.sync_copy(idx_hbm.at[pl.ds(base, per_w)], idx_v)

        vocab_c = jnp.full((16,), _VOCAB, jnp.int32)

        def split_body(i, carry):
            r = idx_v[pl.ds(i * 16, 16)]
            word_v[pl.ds(i * 16, 16)] = lax.rem(r, vocab_c)
            posi_v[pl.ds(i * 16, 16)] = lax.div(r, vocab_c)
            return carry

        lax.fori_loop(0, per_w // 16, split_body, 0)

        def round_body(r, carry):
            descs = []
            for b in range(n_dma):
                off = r * _CHUNK + b * _ROWS_PER_DMA
                dst_rows = pl.ds(b * _ROWS_PER_DMA, _ROWS_PER_DMA)
                descs.append(pltpu.async_copy(
                    emb_hbm.at[word_v.at[pl.ds(off, _ROWS_PER_DMA)]],
                    we_v.at[dst_rows],
                    sem_w))
                descs.append(pltpu.async_copy(
                    pos_hbm.at[posi_v.at[pl.ds(off, _ROWS_PER_DMA)]],
                    pe_v.at[dst_rows],
                    sem_p))
            for d in descs:
                d.wait()
            pltpu.sync_copy(we_v, we_out.at[pl.ds(base + r * _CHUNK, _CHUNK)])
            pltpu.sync_copy(pe_v, pe_out.at[pl.ds(base + r * _CHUNK, _CHUNK)])
            return carry

        lax.fori_loop(0, n_rounds, round_body, 0)

    return gather_kernel(idx_flat, emb_table, pos_pad)


def _blockdiag(a, copies, out_rows_per_copy, out_cols_per_copy):
    rows = []
    for q in range(copies):
        cols = []
        left = q * out_cols_per_copy
        right = (copies - 1 - q) * out_cols_per_copy
        if left:
            cols.append(jnp.zeros((out_rows_per_copy, left), jnp.float32))
        cols.append(a)
        if right:
            cols.append(jnp.zeros((out_rows_per_copy, right), jnp.float32))
        rows.append(jnp.concatenate(cols, axis=1))
    return jnp.concatenate(rows, axis=0)


def _mlp_packed(we4, pe4, W1, b1, W2, b2, block):
    n4 = we4.shape[0]

    def body(we_ref, pe_ref, w1_ref, b1_ref, w2_ref, b2_ref, out_ref):
        w1 = w1_ref[...]
        w2 = w2_ref[...]
        w1a = w1[:_D_EMBED, :]
        w1bp = jnp.concatenate(
            [w1[_D_EMBED:, :], jnp.zeros((_D_POS, _D_EMBED), jnp.float32)],
            axis=0)
        d4a = _blockdiag(w1a, 4, _D_EMBED, _D_EMBED)
        d4b = _blockdiag(w1bp, 4, _D_EMBED, _D_EMBED)
        q4 = _blockdiag(w2, 4, _D_EMBED, 64)
        b1x4 = jnp.concatenate([b1_ref[...]] * 4, axis=1)
        b2x4 = jnp.concatenate([b2_ref[...]] * 4, axis=1)

        h = jnp.dot(we_ref[...], d4a, preferred_element_type=jnp.float32)
        h += jnp.dot(pe_ref[...], d4b, preferred_element_type=jnp.float32)
        h = jnp.maximum(h + b1x4, 0.0)
        out_ref[...] = (
            jnp.dot(h, q4, preferred_element_type=jnp.float32) + b2x4
        )

    return pl.pallas_call(
        body,
        grid=(n4 // block,),
        in_specs=[
            pl.BlockSpec((block, 128), lambda i: (i, 0)),
            pl.BlockSpec((block, 128), lambda i: (i, 0)),
            pl.BlockSpec((48, _D_EMBED), lambda i: (0, 0)),
            pl.BlockSpec((1, _D_EMBED), lambda i: (0, 0)),
            pl.BlockSpec((_D_EMBED, 64), lambda i: (0, 0)),
            pl.BlockSpec((1, 64), lambda i: (0, 0)),
        ],
        out_specs=pl.BlockSpec((block, 256), lambda i: (i, 0)),
        out_shape=jax.ShapeDtypeStruct((n4, 256), jnp.float32),
    )(we4, pe4, W1, b1.reshape(1, -1), W2, b2.reshape(1, -1))


def kernel(inputs, emb_table, pos_table, W1, b1, W2, b2):
    b, l = inputs.shape
    n = b * l
    idx = inputs.reshape(-1).astype(jnp.int32)
    pos_pad = jnp.concatenate(
        [pos_table, jnp.zeros((pos_table.shape[0], _D_POS), jnp.float32)],
        axis=1)
    we, pe = _sc_gather(idx, emb_table, pos_pad)
    we4 = we.reshape(n // 4, 128)
    pe4 = pe.reshape(n // 4, 128)
    out4 = _mlp_packed(we4, pe4, W1, b1, W2, b2, block=1024)
    return out4.reshape(b, l, W2.shape[1])


# speedup vs baseline: 11.1889x; 1.1721x over previous
import functools

import jax
import jax.numpy as jnp
from jax import lax
from jax.experimental import pallas as pl
from jax.experimental.pallas import tpu as pltpu
from jax.experimental.pallas import tpu_sc as plsc

_VOCAB = 1000000
_D_EMBED = 32
_D_POS = 16

_NUM_WORKERS = 32
_ROWS_PER_DMA = 128
_CHUNK = 512


def _sc_gather(idx_flat, emb_table, pos_pad):
    n = idx_flat.shape[0]
    per_w = n // _NUM_WORKERS
    n_rounds = per_w // _CHUNK
    n_dma = _CHUNK // _ROWS_PER_DMA

    mesh = plsc.VectorSubcoreMesh(core_axis_name="c", subcore_axis_name="s")

    @functools.partial(
        pl.kernel,
        mesh=mesh,
        compiler_params=pltpu.CompilerParams(use_tc_tiling_on_sc=False),
        out_type=(
            jax.ShapeDtypeStruct((n, _D_EMBED), jnp.float32),
            jax.ShapeDtypeStruct((n, _D_EMBED), jnp.float32),
        ),
        scratch_types=[
            pltpu.VMEM((per_w,), jnp.int32),
            pltpu.VMEM((per_w,), jnp.int32),
            pltpu.VMEM((per_w,), jnp.int32),
            pltpu.VMEM((2, _CHUNK, _D_EMBED), jnp.float32),
            pltpu.VMEM((2, _CHUNK, _D_EMBED), jnp.float32),
            pltpu.VMEM_SHARED((200, _D_EMBED), jnp.float32),
            pltpu.SemaphoreType.DMA,
            pltpu.SemaphoreType.DMA,
        ],
    )
    def gather_kernel(idx_hbm, emb_hbm, pos_hbm, we_out, pe_out,
                      idx_v, word_v, posi_v, we_v, pe_v, pos_sp,
                      sem_w, sem_p):
        sid = lax.axis_index("s")
        wid = sid * 2 + lax.axis_index("c")
        base = wid * per_w

        @pl.when(sid == 0)
        def _():
            pltpu.sync_copy(pos_hbm, pos_sp)

        pltpu.sync_copy(idx_hbm.at[pl.ds(base, per_w)], idx_v)

        vocab_c = jnp.full((16,), _VOCAB, jnp.int32)

        def split_body(i, carry):
            r = idx_v[pl.ds(i * 16, 16)]
            word_v[pl.ds(i * 16, 16)] = lax.rem(r, vocab_c)
            posi_v[pl.ds(i * 16, 16)] = lax.div(r, vocab_c)
            return carry

        lax.fori_loop(0, per_w // 16, split_body, 0)
        plsc.subcore_barrier()

        def fire(r, buf):
            descs = []
            for b in range(n_dma):
                off = r * _CHUNK + b * _ROWS_PER_DMA
                dst_rows = pl.ds(b * _ROWS_PER_DMA, _ROWS_PER_DMA)
                descs.append(pltpu.async_copy(
                    emb_hbm.at[word_v.at[pl.ds(off, _ROWS_PER_DMA)]],
                    we_v.at[buf].at[dst_rows],
                    sem_w))
                descs.append(pltpu.async_copy(
                    pos_sp.at[posi_v.at[pl.ds(off, _ROWS_PER_DMA)]],
                    pe_v.at[buf].at[dst_rows],
                    sem_p))
            return descs

        def copy_out(r, buf):
            pltpu.sync_copy(we_v.at[buf],
                            we_out.at[pl.ds(base + r * _CHUNK, _CHUNK)])
            pltpu.sync_copy(pe_v.at[buf],
                            pe_out.at[pl.ds(base + r * _CHUNK, _CHUNK)])

        def round_body(r, carry):
            buf = lax.rem(r, 2)
            descs = fire(r, buf)
            @pl.when(r > 0)
            def _():
                copy_out(r - 1, 1 - buf)
            for d in descs:
                d.wait()
            return carry

        lax.fori_loop(0, n_rounds, round_body, 0)
        copy_out(n_rounds - 1, lax.rem(n_rounds - 1, 2))

    return gather_kernel(idx_flat, emb_table, pos_pad)


def _blockdiag(a, copies, out_rows_per_copy, out_cols_per_copy):
    rows = []
    for q in range(copies):
        cols = []
        left = q * out_cols_per_copy
        right = (copies - 1 - q) * out_cols_per_copy
        if left:
            cols.append(jnp.zeros((out_rows_per_copy, left), jnp.float32))
        cols.append(a)
        if right:
            cols.append(jnp.zeros((out_rows_per_copy, right), jnp.float32))
        rows.append(jnp.concatenate(cols, axis=1))
    return jnp.concatenate(rows, axis=0)


def _mlp_packed(we4, pe4, W1, b1, W2, b2, block):
    n4 = we4.shape[0]

    def body(we_ref, pe_ref, w1_ref, b1_ref, w2_ref, b2_ref, out_ref):
        w1 = w1_ref[...]
        w2 = w2_ref[...]
        w1a = w1[:_D_EMBED, :]
        w1bp = jnp.concatenate(
            [w1[_D_EMBED:, :], jnp.zeros((_D_POS, _D_EMBED), jnp.float32)],
            axis=0)
        d4a = _blockdiag(w1a, 4, _D_EMBED, _D_EMBED)
        d4b = _blockdiag(w1bp, 4, _D_EMBED, _D_EMBED)
        q4 = _blockdiag(w2, 4, _D_EMBED, 64)
        b1x4 = jnp.concatenate([b1_ref[...]] * 4, axis=1)
        b2x4 = jnp.concatenate([b2_ref[...]] * 4, axis=1)

        h = jnp.dot(we_ref[...], d4a, preferred_element_type=jnp.float32)
        h += jnp.dot(pe_ref[...], d4b, preferred_element_type=jnp.float32)
        h = jnp.maximum(h + b1x4, 0.0)
        out_ref[...] = (
            jnp.dot(h, q4, preferred_element_type=jnp.float32) + b2x4
        )

    return pl.pallas_call(
        body,
        grid=(n4 // block,),
        in_specs=[
            pl.BlockSpec((block, 128), lambda i: (i, 0)),
            pl.BlockSpec((block, 128), lambda i: (i, 0)),
            pl.BlockSpec((48, _D_EMBED), lambda i: (0, 0)),
            pl.BlockSpec((1, _D_EMBED), lambda i: (0, 0)),
            pl.BlockSpec((_D_EMBED, 64), lambda i: (0, 0)),
            pl.BlockSpec((1, 64), lambda i: (0, 0)),
        ],
        out_specs=pl.BlockSpec((block, 256), lambda i: (i, 0)),
        out_shape=jax.ShapeDtypeStruct((n4, 256), jnp.float32),
    )(we4, pe4, W1, b1.reshape(1, -1), W2, b2.reshape(1, -1))


def kernel(inputs, emb_table, pos_table, W1, b1, W2, b2):
    b, l = inputs.shape
    n = b * l
    idx = inputs.reshape(-1).astype(jnp.int32)
    pos_pad = jnp.concatenate(
        [pos_table, jnp.zeros((pos_table.shape[0], _D_POS), jnp.float32)],
        axis=1)
    we, pe = _sc_gather(idx, emb_table, pos_pad)
    we4 = we.reshape(n // 4, 128)
    pe4 = pe.reshape(n // 4, 128)
    out4 = _mlp_packed(we4, pe4, W1, b1, W2, b2, block=1024)
    return out4.reshape(b, l, W2.shape[1])


# speedup vs baseline: 13.4551x; 1.2025x over previous
import functools

import jax
import jax.numpy as jnp
from jax import lax
from jax.experimental import pallas as pl
from jax.experimental.pallas import tpu as pltpu
from jax.experimental.pallas import tpu_sc as plsc

_VOCAB = 1000000
_D_EMBED = 32
_D_POS = 16
_L = 20
_B = 16384

_NUM_WORKERS = 32
_ROWS_PER_DMA = 128
_CHUNK = 512


def _sc_gather(idx_flat, emb_table, pos_pad):
    n = idx_flat.shape[0]
    per_w = n // _NUM_WORKERS
    n_rounds = per_w // _CHUNK
    n_dma = _CHUNK // _ROWS_PER_DMA
    n_rows = per_w // _ROWS_PER_DMA

    mesh = plsc.VectorSubcoreMesh(core_axis_name="c", subcore_axis_name="s")

    @functools.partial(
        pl.kernel,
        mesh=mesh,
        compiler_params=pltpu.CompilerParams(use_tc_tiling_on_sc=False),
        out_type=(
            jax.ShapeDtypeStruct((n, _D_EMBED), jnp.float32),
            jax.ShapeDtypeStruct((n, _D_EMBED), jnp.float32),
        ),
        scratch_types=[
            pltpu.VMEM((per_w,), jnp.int32),
            pltpu.VMEM((n_rows, _ROWS_PER_DMA), jnp.int32),
            pltpu.VMEM((n_rows, _ROWS_PER_DMA), jnp.int32),
            pltpu.VMEM((n_rows, _ROWS_PER_DMA), jnp.int32),
            pltpu.VMEM((2, _CHUNK, _D_EMBED), jnp.float32),
            pltpu.VMEM((2, _CHUNK, _D_EMBED), jnp.float32),
            pltpu.VMEM_SHARED((200, _D_EMBED), jnp.float32),
            pltpu.SemaphoreType.DMA,
            pltpu.SemaphoreType.DMA,
            pltpu.SemaphoreType.DMA,
        ],
    )
    def gather_kernel(idx_hbm, emb_hbm, pos_hbm, we_out, pe_out,
                      idx_v, word_v, posi_v, prow_v, we_v, pe_v, pos_sp,
                      sem_w, sem_p, sem_o):
        sid = lax.axis_index("s")
        wid = sid * 2 + lax.axis_index("c")
        base = wid * per_w

        @pl.when(sid == 0)
        def _():
            pltpu.sync_copy(pos_hbm, pos_sp)

        pltpu.sync_copy(idx_hbm.at[pl.ds(base, per_w)], idx_v)

        vocab_c = jnp.full((16,), _VOCAB, jnp.int32)
        l_c = jnp.full((16,), _L, jnp.int32)
        b_c = jnp.full((16,), _B, jnp.int32)
        lane = lax.iota(jnp.int32, 16)

        def split_body(j, carry):
            row = word_v.at[j]
            prow = posi_v.at[j]
            srow = prow_v.at[j]
            for s in range(_ROWS_PER_DMA // 16):
                sl = pl.ds(s * 16, 16)
                r = idx_v[pl.ds(j * _ROWS_PER_DMA + s * 16, 16)]
                row[sl] = lax.rem(r, vocab_c)
                prow[sl] = lax.div(r, vocab_c)
                t = (base + j * _ROWS_PER_DMA + s * 16) + lane
                bb = lax.div(t, l_c)
                c = jnp.bitwise_and(bb, 2047)
                srow[sl] = (
                    lax.rem(t, l_c) * b_c
                    + jnp.bitwise_and(bb, jnp.int32(~2047))
                    + lax.shift_left(jnp.bitwise_and(c, 511), 2)
                    + lax.shift_right_logical(c, 9))
            return carry

        lax.fori_loop(0, n_rows, split_body, 0)
        plsc.subcore_barrier()

        def fire(r, buf):
            descs = []
            for b in range(n_dma):
                jj = r * n_dma + b
                dst_rows = pl.ds(b * _ROWS_PER_DMA, _ROWS_PER_DMA)
                descs.append(pltpu.async_copy(
                    emb_hbm.at[word_v.at[jj]],
                    we_v.at[buf].at[dst_rows],
                    sem_w))
                descs.append(pltpu.async_copy(
                    pos_sp.at[posi_v.at[jj]],
                    pe_v.at[buf].at[dst_rows],
                    sem_p))
            return descs

        def scatter_out(r, buf):
            descs = []
            for b in range(n_dma):
                jj = r * n_dma + b
                src_rows = pl.ds(b * _ROWS_PER_DMA, _ROWS_PER_DMA)
                descs.append(pltpu.async_copy(
                    we_v.at[buf].at[src_rows],
                    we_out.at[prow_v.at[jj]],
                    sem_o))
                descs.append(pltpu.async_copy(
                    pe_v.at[buf].at[src_rows],
                    pe_out.at[prow_v.at[jj]],
                    sem_o))
            return descs

        def round_body(r, carry):
            buf = lax.rem(r, 2)
            g = fire(r, buf)
            @pl.when(r > 0)
            def _():
                for d in scatter_out(r - 1, 1 - buf):
                    d.wait()
            for d in g:
                d.wait()
            return carry

        lax.fori_loop(0, n_rounds, round_body, 0)
        for d in scatter_out(n_rounds - 1, lax.rem(n_rounds - 1, 2)):
            d.wait()

    return gather_kernel(idx_flat, emb_table, pos_pad)


def _blockdiag(a, copies, out_rows_per_copy, out_cols_per_copy):
    rows = []
    for q in range(copies):
        cols = []
        left = q * out_cols_per_copy
        right = (copies - 1 - q) * out_cols_per_copy
        if left:
            cols.append(jnp.zeros((out_rows_per_copy, left), jnp.float32))
        cols.append(a)
        if right:
            cols.append(jnp.zeros((out_rows_per_copy, right), jnp.float32))
        rows.append(jnp.concatenate(cols, axis=1))
    return jnp.concatenate(rows, axis=0)


def _mlp_packed_t(we4, pe4, W1, b1, W2, b2, block, bs):
    n4 = we4.shape[0]
    n_bc = _B // bs

    def body(we_ref, pe_ref, w1_ref, b1_ref, w2t_ref, b2_ref, out_ref):
        w1 = w1_ref[...]
        w1a = w1[:_D_EMBED, :]
        w1bp = jnp.concatenate(
            [w1[_D_EMBED:, :], jnp.zeros((_D_POS, _D_EMBED), jnp.float32)],
            axis=0)
        d4a = _blockdiag(w1a, 4, _D_EMBED, _D_EMBED)
        d4b = _blockdiag(w1bp, 4, _D_EMBED, _D_EMBED)
        b1x4 = jnp.concatenate([b1_ref[...]] * 4, axis=1)

        h = jnp.dot(we_ref[...], d4a, preferred_element_type=jnp.float32)
        h += jnp.dot(pe_ref[...], d4b, preferred_element_type=jnp.float32)
        h = jnp.maximum(h + b1x4, 0.0)
        w2t = w2t_ref[...]
        parts = [
            lax.dot_general(
                w2t, h[:, q * _D_EMBED:(q + 1) * _D_EMBED],
                (((1,), (1,)), ((), ())),
                preferred_element_type=jnp.float32)
            for q in range(4)
        ]
        out_ref[...] = jnp.concatenate(parts, axis=1) + b2_ref[...]

    return pl.pallas_call(
        body,
        grid=(n4 // block,),
        in_specs=[
            pl.BlockSpec((block, 128), lambda i: (i, 0)),
            pl.BlockSpec((block, 128), lambda i: (i, 0)),
            pl.BlockSpec((48, _D_EMBED), lambda i: (0, 0)),
            pl.BlockSpec((1, _D_EMBED), lambda i: (0, 0)),
            pl.BlockSpec((64, _D_EMBED), lambda i: (0, 0)),
            pl.BlockSpec((64, 1), lambda i: (0, 0)),
        ],
        out_specs=pl.BlockSpec(
            (64, bs), lambda i: (i // n_bc, i % n_bc)),
        out_shape=jax.ShapeDtypeStruct((_L * 64, _B), jnp.float32),
    )(we4, pe4, W1, b1.reshape(1, -1), jnp.swapaxes(W2, 0, 1),
      b2.reshape(-1, 1))


def kernel(inputs, emb_table, pos_table, W1, b1, W2, b2):
    b, l = inputs.shape
    n = b * l
    idx = inputs.reshape(-1).astype(jnp.int32)
    pos_pad = jnp.concatenate(
        [pos_table, jnp.zeros((pos_table.shape[0], _D_POS), jnp.float32)],
        axis=1)
    we, pe = _sc_gather(idx, emb_table, pos_pad)
    we4 = we.reshape(n // 4, 128)
    pe4 = pe.reshape(n // 4, 128)
    bs = 2048
    out2d = _mlp_packed_t(we4, pe4, W1, b1, W2, b2, block=bs // 4, bs=bs)
    return out2d.reshape(l, 64, b).transpose(2, 0, 1)


# speedup vs baseline: 14.4384x; 1.0731x over previous
import functools

import jax
import jax.numpy as jnp
from jax import lax
from jax.experimental import pallas as pl
from jax.experimental.pallas import tpu as pltpu
from jax.experimental.pallas import tpu_sc as plsc

_VOCAB = 1000000
_D_EMBED = 32
_D_POS = 16
_L = 20
_B = 16384

_NUM_WORKERS = 32
_ROWS_PER_DMA = 128
_CHUNK = 512


def _sc_gather(idx_flat, emb_table, pos_pad):
    n = idx_flat.shape[0]
    per_w = n // _NUM_WORKERS
    n_rounds = per_w // _CHUNK
    n_dma = _CHUNK // _ROWS_PER_DMA
    n_rows = per_w // _ROWS_PER_DMA

    mesh = plsc.VectorSubcoreMesh(core_axis_name="c", subcore_axis_name="s")

    @functools.partial(
        pl.kernel,
        mesh=mesh,
        compiler_params=pltpu.CompilerParams(use_tc_tiling_on_sc=False),
        out_type=(
            jax.ShapeDtypeStruct((n, _D_EMBED), jnp.float32),
            jax.ShapeDtypeStruct((n, _D_EMBED), jnp.float32),
        ),
        scratch_types=[
            pltpu.VMEM((per_w,), jnp.int32),
            pltpu.VMEM((n_rows, _ROWS_PER_DMA), jnp.int32),
            pltpu.VMEM((n_rows, _ROWS_PER_DMA), jnp.int32),
            pltpu.VMEM((n_rows, _ROWS_PER_DMA), jnp.int32),
            pltpu.VMEM((2, _CHUNK, _D_EMBED), jnp.float32),
            pltpu.VMEM((2, _CHUNK, _D_EMBED), jnp.float32),
            pltpu.VMEM_SHARED((200, _D_EMBED), jnp.float32),
            pltpu.SemaphoreType.DMA,
            pltpu.SemaphoreType.DMA,
            pltpu.SemaphoreType.DMA,
        ],
    )
    def gather_kernel(idx_hbm, emb_hbm, pos_hbm, we_out, pe_out,
                      idx_v, word_v, posi_v, prow_v, we_v, pe_v, pos_sp,
                      sem_w, sem_p, sem_o):
        sid = lax.axis_index("s")
        wid = sid * 2 + lax.axis_index("c")
        base = wid * per_w

        @pl.when(sid == 0)
        def _():
            pltpu.sync_copy(pos_hbm, pos_sp)

        pltpu.sync_copy(idx_hbm.at[pl.ds(base, per_w)], idx_v)

        vocab_c = jnp.full((16,), _VOCAB, jnp.int32)
        l_c = jnp.full((16,), _L, jnp.int32)
        b_c = jnp.full((16,), _B, jnp.int32)
        lane = lax.iota(jnp.int32, 16)

        def split_body(j, carry):
            row = word_v.at[j]
            prow = posi_v.at[j]
            srow = prow_v.at[j]
            for s in range(_ROWS_PER_DMA // 16):
                sl = pl.ds(s * 16, 16)
                r = idx_v[pl.ds(j * _ROWS_PER_DMA + s * 16, 16)]
                row[sl] = lax.rem(r, vocab_c)
                prow[sl] = lax.div(r, vocab_c)
                t = (base + j * _ROWS_PER_DMA + s * 16) + lane
                bb = lax.div(t, l_c)
                c = jnp.bitwise_and(bb, 2047)
                srow[sl] = (
                    lax.rem(t, l_c) * b_c
                    + jnp.bitwise_and(bb, jnp.int32(~2047))
                    + lax.shift_left(jnp.bitwise_and(c, 511), 2)
                    + lax.shift_right_logical(c, 9))
            return carry

        lax.fori_loop(0, n_rows, split_body, 0)
        plsc.subcore_barrier()

        def fire(r, buf):
            descs = []
            for b in range(n_dma):
                jj = r * n_dma + b
                dst_rows = pl.ds(b * _ROWS_PER_DMA, _ROWS_PER_DMA)
                descs.append(pltpu.async_copy(
                    emb_hbm.at[word_v.at[jj]],
                    we_v.at[buf].at[dst_rows],
                    sem_w))
                descs.append(pltpu.async_copy(
                    pos_sp.at[posi_v.at[jj]],
                    pe_v.at[buf].at[dst_rows],
                    sem_p))
            return descs

        def scatter_out(r, buf):
            descs = []
            for b in range(n_dma):
                jj = r * n_dma + b
                src_rows = pl.ds(b * _ROWS_PER_DMA, _ROWS_PER_DMA)
                descs.append(pltpu.async_copy(
                    we_v.at[buf].at[src_rows],
                    we_out.at[prow_v.at[jj]],
                    sem_o))
                descs.append(pltpu.async_copy(
                    pe_v.at[buf].at[src_rows],
                    pe_out.at[prow_v.at[jj]],
                    sem_o))
            return descs

        def round_body(r, carry):
            buf = lax.rem(r, 2)
            g = fire(r, buf)
            @pl.when(r > 0)
            def _():
                for d in scatter_out(r - 1, 1 - buf):
                    d.wait()
            for d in g:
                d.wait()
            return carry

        lax.fori_loop(0, n_rounds, round_body, 0)
        for d in scatter_out(n_rounds - 1, lax.rem(n_rounds - 1, 2)):
            d.wait()

    return gather_kernel(idx_flat, emb_table, pos_pad)


def _blockdiag(a, copies, out_rows_per_copy, out_cols_per_copy):
    rows = []
    for q in range(copies):
        cols = []
        left = q * out_cols_per_copy
        right = (copies - 1 - q) * out_cols_per_copy
        if left:
            cols.append(jnp.zeros((out_rows_per_copy, left), jnp.float32))
        cols.append(a)
        if right:
            cols.append(jnp.zeros((out_rows_per_copy, right), jnp.float32))
        rows.append(jnp.concatenate(cols, axis=1))
    return jnp.concatenate(rows, axis=0)


def _mlp_packed_t(we4, pe4, W1, b1, W2, b2, block, bs):
    n4 = we4.shape[0]
    n_bc = _B // bs

    def body(we_ref, pe_ref, w1_ref, b1_ref, w2t_ref, b2_ref, out_ref):
        w1 = w1_ref[...]
        w1a = w1[:_D_EMBED, :]
        w1bp = jnp.concatenate(
            [w1[_D_EMBED:, :], jnp.zeros((_D_POS, _D_EMBED), jnp.float32)],
            axis=0)
        d4a = _blockdiag(w1a, 4, _D_EMBED, _D_EMBED)
        d4b = _blockdiag(w1bp, 4, _D_EMBED, _D_EMBED)
        b1x4 = jnp.concatenate([b1_ref[...]] * 4, axis=1)

        h = jnp.dot(we_ref[...], d4a, preferred_element_type=jnp.float32)
        h += jnp.dot(pe_ref[...], d4b, preferred_element_type=jnp.float32)
        h = jnp.maximum(h + b1x4, 0.0)
        w2t = w2t_ref[...]
        parts = [
            lax.dot_general(
                w2t, h[:, q * _D_EMBED:(q + 1) * _D_EMBED],
                (((1,), (1,)), ((), ())),
                preferred_element_type=jnp.float32)
            for q in range(4)
        ]
        out_ref[...] = jnp.concatenate(parts, axis=1) + b2_ref[...]

    return pl.pallas_call(
        body,
        grid=(n4 // block,),
        in_specs=[
            pl.BlockSpec((block, 128), lambda i: (i, 0)),
            pl.BlockSpec((block, 128), lambda i: (i, 0)),
            pl.BlockSpec((48, _D_EMBED), lambda i: (0, 0)),
            pl.BlockSpec((1, _D_EMBED), lambda i: (0, 0)),
            pl.BlockSpec((64, _D_EMBED), lambda i: (0, 0)),
            pl.BlockSpec((64, 1), lambda i: (0, 0)),
        ],
        out_specs=pl.BlockSpec(
            (64, bs), lambda i: (i // n_bc, i % n_bc)),
        out_shape=jax.ShapeDtypeStruct((_L * 64, _B), jnp.float32),
    )(we4, pe4, W1, b1.reshape(1, -1), jnp.swapaxes(W2, 0, 1),
      b2.reshape(-1, 1))


def kernel(inputs, emb_table, pos_table, W1, b1, W2, b2):
    b, l = inputs.shape
    n = b * l
    idx = inputs.reshape(-1).astype(jnp.int32)
    pos_pad = jnp.concatenate(
        [pos_table, jnp.zeros((pos_table.shape[0], _D_POS), jnp.float32)],
        axis=1)
    we, pe = _sc_gather(idx, emb_table, pos_pad)
    we4 = we.reshape(n // 4, 128)
    pe4 = pe.reshape(n // 4, 128)
    bs = 4096
    out2d = _mlp_packed_t(we4, pe4, W1, b1, W2, b2, block=bs // 4, bs=bs)
    return out2d.reshape(l, 64, b).transpose(2, 0, 1)
